# baseline (speedup 1.0000x reference)
import functools
import math

import jax
import jax.numpy as jnp
from jax import lax
from jax.experimental import pallas as pl
from jax.experimental.pallas import tpu as pltpu
from jax.experimental.pallas import tpu_sc as plsc

B = 16384
DATA_DIM = 512
DIM = 64
HALF = DIM // 2
NUM_CELL = 100

_SIN_C = (6.283168273564918, -41.337929774906165, 81.47313282270473,
          -75.0932740471627, 33.95650071282797)
_COS_C = (0.9999994434755294, -19.739034355263385, 64.93061294590028,
          -85.29596684284616, 58.91253793524945, -21.282995036331283)


def _sincos_2pi(u):
    r = u - jnp.floor(u + 0.5)
    z = r * r
    s = _SIN_C[4]
    for c in (_SIN_C[3], _SIN_C[2], _SIN_C[1], _SIN_C[0]):
        s = s * z + c
    s = s * r
    q = _COS_C[5]
    for c in (_COS_C[4], _COS_C[3], _COS_C[2], _COS_C[1], _COS_C[0]):
        q = q * z + c
    return s, q


_BS = 2048


def _tc_body(t_ref, xt_ref, cell_ref, dose_ref, tf_ref, df_ref, wp_ref,
             bp_ref, ct_ref, time_out, xt_out, cell_out, dose_out):
    u = tf_ref[...] * t_ref[...]
    s, c = _sincos_2pi(u)
    time_out[0:HALF, :] = s
    time_out[HALF:DIM, :] = c

    xt_out[...] = lax.dot_general(
        wp_ref[...], xt_ref[...], (((0,), (1,)), ((), ())),
        preferred_element_type=jnp.float32) + bp_ref[...]

    idx = cell_ref[...]
    iota = lax.broadcasted_iota(jnp.int32, (NUM_CELL, _BS), 0)
    onehot = (iota == idx).astype(jnp.float32)
    cell_out[...] = lax.dot_general(
        ct_ref[...], onehot, (((0,), (0,)), ((), ())),
        preferred_element_type=jnp.float32)

    for p in range(3):
        up = df_ref[...] * dose_ref[pl.ds(p, 1), :]
        sp, cp = _sincos_2pi(up)
        dose_out[pl.ds(DIM * p, HALF), :] = sp
        dose_out[pl.ds(DIM * p + HALF, HALF), :] = cp


def _tc_dense(time, xt, cell_type, dose3, time_freqs, dose_freqs, Wp, bp,
              cell_table):
    grid = (B // _BS,)
    return pl.pallas_call(
        _tc_body,
        grid=grid,
        in_specs=[
            pl.BlockSpec((1, _BS), lambda i: (0, i)),
            pl.BlockSpec((_BS, DATA_DIM), lambda i: (i, 0)),
            pl.BlockSpec((1, _BS), lambda i: (0, i)),
            pl.BlockSpec((3, _BS), lambda i: (0, i)),
            pl.BlockSpec((HALF, 1), lambda i: (0, 0)),
            pl.BlockSpec((HALF, 1), lambda i: (0, 0)),
            pl.BlockSpec((DATA_DIM, DIM), lambda i: (0, 0)),
            pl.BlockSpec((DIM, 1), lambda i: (0, 0)),
            pl.BlockSpec((NUM_CELL, DIM), lambda i: (0, 0)),
        ],
        out_specs=[
            pl.BlockSpec((DIM, _BS), lambda i: (0, i)),
            pl.BlockSpec((DIM, _BS), lambda i: (0, i)),
            pl.BlockSpec((DIM, _BS), lambda i: (0, i)),
            pl.BlockSpec((3 * DIM, _BS), lambda i: (0, i)),
        ],
        out_shape=[
            jax.ShapeDtypeStruct((DIM, B), jnp.float32),
            jax.ShapeDtypeStruct((DIM, B), jnp.float32),
            jax.ShapeDtypeStruct((DIM, B), jnp.float32),
            jax.ShapeDtypeStruct((3 * DIM, B), jnp.float32),
        ],
    )(
        time.reshape(1, B), xt, cell_type.reshape(1, B), dose3,
        time_freqs.reshape(HALF, 1), dose_freqs.reshape(HALF, 1), Wp,
        bp.reshape(DIM, 1), cell_table,
    )


_CHUNK = 128
_N = 3 * B


def _sc_gather_both(gene_table, mol_table, gene_idx, mol_idx):
    info = plsc.get_sparse_core_info()
    nw = info.num_cores * info.num_subcores
    per_w = _N // nw
    n_ch = per_w // _CHUNK
    assert per_w % _CHUNK == 0
    gidx3 = gene_idx.reshape(nw, n_ch, _CHUNK)
    midx3 = mol_idx.reshape(nw, n_ch, _CHUNK)
    mesh = plsc.VectorSubcoreMesh(core_axis_name="c", subcore_axis_name="s")

    def body(gidx_hbm, midx_hbm, gtab_hbm, mtab_hbm, gout_hbm, mout_hbm,
             giv, miv, buf0, buf1, sem0, sem1):
        wid = lax.axis_index("s") * info.num_cores + lax.axis_index("c")
        pltpu.sync_copy(gidx_hbm.at[wid], giv)
        pltpu.sync_copy(midx_hbm.at[wid], miv)
        for idx_v, tab, out in ((giv, gtab_hbm, gout_hbm),
                                (miv, mtab_hbm, mout_hbm)):
            bufs = (buf0, buf1)
            sems = (sem0, sem1)
            cps = [None] * n_ch
            cps[0] = pltpu.async_copy(tab.at[idx_v.at[0]], buf0, sem0)
            for j in range(n_ch):
                if j + 1 < n_ch:
                    cps[j + 1] = pltpu.async_copy(
                        tab.at[idx_v.at[j + 1]], bufs[(j + 1) % 2],
                        sems[(j + 1) % 2])
                cps[j].wait()
                pltpu.sync_copy(
                    bufs[j % 2],
                    out.at[pl.ds(wid * per_w + j * _CHUNK, _CHUNK)])

    fn = pl.kernel(
        body,
        out_type=(jax.ShapeDtypeStruct((_N, DIM), jnp.float32),
                  jax.ShapeDtypeStruct((_N, DIM), jnp.float32)),
        mesh=mesh,
        scratch_types=[
            pltpu.VMEM((n_ch, _CHUNK), jnp.int32),
            pltpu.VMEM((n_ch, _CHUNK), jnp.int32),
            pltpu.VMEM((_CHUNK, DIM), jnp.float32),
            pltpu.VMEM((_CHUNK, DIM), jnp.float32),
            pltpu.SemaphoreType.DMA,
            pltpu.SemaphoreType.DMA,
        ],
        compiler_params=pltpu.CompilerParams(use_tc_tiling_on_sc=False),
    )
    return fn(gidx3, midx3, gene_table, mol_table)


_TBS = 2048


def _tr_body(g_ref, m_ref, g_out, m_out):
    g_out[...] = jnp.transpose(g_ref[...])[None]
    m_out[...] = jnp.transpose(m_ref[...])[None]


def _tc_transpose(gene_rows, mol_rows):
    nb = B // _TBS
    grid = (_N // _TBS,)
    return pl.pallas_call(
        _tr_body,
        grid=grid,
        in_specs=[
            pl.BlockSpec((_TBS, DIM), lambda i: (i, 0)),
            pl.BlockSpec((_TBS, DIM), lambda i: (i, 0)),
        ],
        out_specs=[
            pl.BlockSpec((1, DIM, _TBS), lambda i: (i // nb, 0, i % nb)),
            pl.BlockSpec((1, DIM, _TBS), lambda i: (i // nb, 0, i % nb)),
        ],
        out_shape=[
            jax.ShapeDtypeStruct((3, DIM, B), jnp.float32),
            jax.ShapeDtypeStruct((3, DIM, B), jnp.float32),
        ],
    )(gene_rows, mol_rows)


def kernel(time, xt, cell_type, gene_pert_idx, mol_pert_idx, mol_dose,
           time_freqs, dose_freqs, Wp, bp, cell_table, gene_table, mol_table):
    dose3 = mol_dose.reshape(3, B)
    time_T, xt_T, cell_T, dose_T = _tc_dense(
        time, xt, cell_type, dose3, time_freqs, dose_freqs, Wp, bp,
        cell_table)

    gene_flat, mol_flat = _sc_gather_both(
        gene_table, mol_table, gene_pert_idx.reshape(-1),
        mol_pert_idx.reshape(-1))
    gene_T, mol_T = _tc_transpose(gene_flat, mol_flat)

    time_emb = time_T.T
    xt_emb = xt_T.T
    cell_emb = cell_T.T
    dose_emb = dose_T.reshape(3, DIM, B).transpose(0, 2, 1)
    gene_emb = gene_T.transpose(0, 2, 1)
    mol_emb = mol_T.transpose(0, 2, 1)
    return (time_emb, xt_emb, cell_emb, gene_emb, mol_emb, dose_emb)


# speedup vs baseline: 1.8286x; 1.8286x over previous
import functools
import math

import jax
import jax.numpy as jnp
from jax import lax
from jax.experimental import pallas as pl
from jax.experimental.pallas import tpu as pltpu
from jax.experimental.pallas import tpu_sc as plsc

B = 16384
DATA_DIM = 512
DIM = 64
HALF = DIM // 2
NUM_CELL = 100

_SIN_C = (6.283168273564918, -41.337929774906165, 81.47313282270473,
          -75.0932740471627, 33.95650071282797)
_COS_C = (0.9999994434755294, -19.739034355263385, 64.93061294590028,
          -85.29596684284616, 58.91253793524945, -21.282995036331283)


def _sincos_2pi(u):
    r = u - jnp.floor(u + 0.5)
    z = r * r
    s = _SIN_C[4]
    for c in (_SIN_C[3], _SIN_C[2], _SIN_C[1], _SIN_C[0]):
        s = s * z + c
    s = s * r
    q = _COS_C[5]
    for c in (_COS_C[4], _COS_C[3], _COS_C[2], _COS_C[1], _COS_C[0]):
        q = q * z + c
    return s, q


_BS = 2048


def _tc_body(t_ref, xt_ref, cell_ref, dose_ref, tf_ref, df_ref, wp_ref,
             bp_ref, ct_ref, time_out, xt_out, cell_out, dose_out):
    u = tf_ref[...] * t_ref[...]
    s, c = _sincos_2pi(u)
    time_out[0:HALF, :] = s
    time_out[HALF:DIM, :] = c

    xt_out[...] = lax.dot_general(
        wp_ref[...], xt_ref[...], (((0,), (1,)), ((), ())),
        preferred_element_type=jnp.float32) + bp_ref[...]

    idx = cell_ref[...]
    iota = lax.broadcasted_iota(jnp.int32, (NUM_CELL, _BS), 0)
    onehot = (iota == idx).astype(jnp.float32)
    cell_out[...] = lax.dot_general(
        ct_ref[...], onehot, (((0,), (0,)), ((), ())),
        preferred_element_type=jnp.float32)

    for p in range(3):
        up = df_ref[...] * dose_ref[pl.ds(p, 1), :]
        sp, cp = _sincos_2pi(up)
        dose_out[pl.ds(DIM * p, HALF), :] = sp
        dose_out[pl.ds(DIM * p + HALF, HALF), :] = cp


def _tc_dense(time, xt, cell_type, dose3, time_freqs, dose_freqs, Wp, bp,
              cell_table):
    grid = (B // _BS,)
    return pl.pallas_call(
        _tc_body,
        grid=grid,
        in_specs=[
            pl.BlockSpec((1, _BS), lambda i: (0, i)),
            pl.BlockSpec((_BS, DATA_DIM), lambda i: (i, 0)),
            pl.BlockSpec((1, _BS), lambda i: (0, i)),
            pl.BlockSpec((3, _BS), lambda i: (0, i)),
            pl.BlockSpec((HALF, 1), lambda i: (0, 0)),
            pl.BlockSpec((HALF, 1), lambda i: (0, 0)),
            pl.BlockSpec((DATA_DIM, DIM), lambda i: (0, 0)),
            pl.BlockSpec((DIM, 1), lambda i: (0, 0)),
            pl.BlockSpec((NUM_CELL, DIM), lambda i: (0, 0)),
        ],
        out_specs=[
            pl.BlockSpec((DIM, _BS), lambda i: (0, i)),
            pl.BlockSpec((DIM, _BS), lambda i: (0, i)),
            pl.BlockSpec((DIM, _BS), lambda i: (0, i)),
            pl.BlockSpec((3 * DIM, _BS), lambda i: (0, i)),
        ],
        out_shape=[
            jax.ShapeDtypeStruct((DIM, B), jnp.float32),
            jax.ShapeDtypeStruct((DIM, B), jnp.float32),
            jax.ShapeDtypeStruct((DIM, B), jnp.float32),
            jax.ShapeDtypeStruct((3 * DIM, B), jnp.float32),
        ],
    )(
        time.reshape(1, B), xt, cell_type.reshape(1, B), dose3,
        time_freqs.reshape(HALF, 1), dose_freqs.reshape(HALF, 1), Wp,
        bp.reshape(DIM, 1), cell_table,
    )


_VB = 8192


def _pad_t_body(t_ref, out_ref):
    out_ref[:, 0:DIM] = jnp.transpose(t_ref[...])


def _tc_pad_transpose(table_t):
    v = table_t.shape[1]
    grid = ((v + _VB - 1) // _VB,)
    return pl.pallas_call(
        _pad_t_body,
        grid=grid,
        in_specs=[pl.BlockSpec((DIM, _VB), lambda i: (0, i))],
        out_specs=pl.BlockSpec((_VB, 128), lambda i: (i, 0)),
        out_shape=jax.ShapeDtypeStruct((v, 128), jnp.float32),
    )(table_t)


_CHUNK = 128
_N = 3 * B


def _sc_gather_both(gene_pad, mol_pad, gene_idx, mol_idx):
    info = plsc.get_sparse_core_info()
    nw = info.num_cores * info.num_subcores
    per_w = _N // nw
    n_ch = per_w // _CHUNK
    assert per_w % _CHUNK == 0
    gidx3 = jnp.pad(gene_idx.reshape(nw, n_ch, _CHUNK), ((0, 0), (0, 4), (0, 0)))
    midx3 = jnp.pad(mol_idx.reshape(nw, n_ch, _CHUNK), ((0, 0), (0, 4), (0, 0)))
    mesh = plsc.VectorSubcoreMesh(core_axis_name="c", subcore_axis_name="s")

    def body(gidx_hbm, midx_hbm, gtab_hbm, mtab_hbm, gout_hbm, mout_hbm,
             giv, miv, buf0, buf1, sem0, sem1):
        wid = lax.axis_index("s") * info.num_cores + lax.axis_index("c")
        base = pl.multiple_of(wid * per_w, _CHUNK)
        pltpu.sync_copy(gidx_hbm.at[wid], giv)
        pltpu.sync_copy(midx_hbm.at[wid], miv)
        for idx_v, tab, out in ((giv, gtab_hbm, gout_hbm),
                                (miv, mtab_hbm, mout_hbm)):
            bufs = (buf0, buf1)
            sems = (sem0, sem1)
            cps = [None] * n_ch
            cps[0] = pltpu.async_copy(tab.at[idx_v.at[0]], buf0, sem0)
            for j in range(n_ch):
                if j + 1 < n_ch:
                    cps[j + 1] = pltpu.async_copy(
                        tab.at[idx_v.at[j + 1]], bufs[(j + 1) % 2],
                        sems[(j + 1) % 2])
                cps[j].wait()
                pltpu.sync_copy(
                    bufs[j % 2], out.at[pl.ds(base + j * _CHUNK, _CHUNK)])

    fn = pl.kernel(
        body,
        out_type=(jax.ShapeDtypeStruct((_N, 128), jnp.float32),
                  jax.ShapeDtypeStruct((_N, 128), jnp.float32)),
        mesh=mesh,
        scratch_types=[
            pltpu.VMEM((n_ch + 4, _CHUNK), jnp.int32),
            pltpu.VMEM((n_ch + 4, _CHUNK), jnp.int32),
            pltpu.VMEM((_CHUNK, 128), jnp.float32),
            pltpu.VMEM((_CHUNK, 128), jnp.float32),
            pltpu.SemaphoreType.DMA,
            pltpu.SemaphoreType.DMA,
        ],
        compiler_params=pltpu.CompilerParams(use_tc_tiling_on_sc=True),
    )
    return fn(gidx3, midx3, gene_pad, mol_pad)


_TBS = 2048


def _tr_body(g_ref, m_ref, g_out, m_out):
    g_out[...] = jnp.transpose(g_ref[:, 0:DIM])[None]
    m_out[...] = jnp.transpose(m_ref[:, 0:DIM])[None]


def _tc_transpose(gene_rows, mol_rows):
    nb = B // _TBS
    grid = (_N // _TBS,)
    return pl.pallas_call(
        _tr_body,
        grid=grid,
        in_specs=[
            pl.BlockSpec((_TBS, 128), lambda i: (i, 0)),
            pl.BlockSpec((_TBS, 128), lambda i: (i, 0)),
        ],
        out_specs=[
            pl.BlockSpec((1, DIM, _TBS), lambda i: (i // nb, 0, i % nb)),
            pl.BlockSpec((1, DIM, _TBS), lambda i: (i // nb, 0, i % nb)),
        ],
        out_shape=[
            jax.ShapeDtypeStruct((3, DIM, B), jnp.float32),
            jax.ShapeDtypeStruct((3, DIM, B), jnp.float32),
        ],
    )(gene_rows, mol_rows)


def kernel(time, xt, cell_type, gene_pert_idx, mol_pert_idx, mol_dose,
           time_freqs, dose_freqs, Wp, bp, cell_table, gene_table, mol_table):
    dose3 = mol_dose.reshape(3, B)
    time_T, xt_T, cell_T, dose_T = _tc_dense(
        time, xt, cell_type, dose3, time_freqs, dose_freqs, Wp, bp,
        cell_table)

    gene_pad = _tc_pad_transpose(gene_table.T)
    mol_pad = _tc_pad_transpose(mol_table.T)
    gene_flat, mol_flat = _sc_gather_both(
        gene_pad, mol_pad, gene_pert_idx.reshape(-1),
        mol_pert_idx.reshape(-1))
    gene_T, mol_T = _tc_transpose(gene_flat, mol_flat)

    time_emb = time_T.T
    xt_emb = xt_T.T
    cell_emb = cell_T.T
    dose_emb = dose_T.reshape(3, DIM, B).transpose(0, 2, 1)
    gene_emb = gene_T.transpose(0, 2, 1)
    mol_emb = mol_T.transpose(0, 2, 1)
    return (time_emb, xt_emb, cell_emb, gene_emb, mol_emb, dose_emb)
